# baseline (speedup 1.0000x reference)
import functools

import jax
import jax.numpy as jnp
from jax import lax
from jax.experimental import pallas as pl
from jax.experimental.pallas import tpu as pltpu
from jax.experimental.pallas import tpu_sc as plsc

DIM_H = 1024
NV = 320
GR = 2
DCODE = 128
TB = 1024


def _argmax_body(hs_ref, w0_ref, w1_ref, b0_ref, b1_ref, idx0_ref, idx1_ref):
    hs = hs_ref[...]
    l0 = jnp.dot(hs, w0_ref[...], preferred_element_type=jnp.float32) + b0_ref[...]
    l1 = jnp.dot(hs, w1_ref[...], preferred_element_type=jnp.float32) + b1_ref[...]
    iota = lax.broadcasted_iota(jnp.int32, (TB, NV), 1)
    big = jnp.int32(2 ** 30)

    m0 = jnp.max(l0, axis=1, keepdims=True)
    i0 = jnp.min(jnp.where(l0 == m0, iota, big), axis=1)
    m1 = jnp.max(l1, axis=1, keepdims=True)
    i1 = jnp.min(jnp.where(l1 == m1, iota, big), axis=1)

    idx0_ref[...] = i0
    idx1_ref[...] = i1 + NV


def _argmax_call(hs2d, w0, w1, b0, b1):
    T = hs2d.shape[0]
    return pl.pallas_call(
        _argmax_body,
        grid=(T // TB,),
        in_specs=[
            pl.BlockSpec((TB, DIM_H), lambda i: (i, 0)),
            pl.BlockSpec((DIM_H, NV), lambda i: (0, 0)),
            pl.BlockSpec((DIM_H, NV), lambda i: (0, 0)),
            pl.BlockSpec((1, NV), lambda i: (0, 0)),
            pl.BlockSpec((1, NV), lambda i: (0, 0)),
        ],
        out_specs=[
            pl.BlockSpec((TB,), lambda i: (i,)),
            pl.BlockSpec((TB,), lambda i: (i,)),
        ],
        out_shape=[
            jax.ShapeDtypeStruct((T,), jnp.int32),
            jax.ShapeDtypeStruct((T,), jnp.int32),
        ],
    )(hs2d, w0, w1, b0, b1)


def _dist_body(idx0_ref, idx1_ref, dist_ref):
    iota = lax.broadcasted_iota(jnp.int32, (NV, TB), 0)
    i0 = idx0_ref[...]
    i1 = idx1_ref[...] - NV
    dist_ref[0] = (iota == i0[None, :]).astype(jnp.float32)
    dist_ref[1] = (iota == i1[None, :]).astype(jnp.float32)


def _dist_call(idx0, idx1):
    T = idx0.shape[0]
    return pl.pallas_call(
        _dist_body,
        grid=(T // TB,),
        in_specs=[
            pl.BlockSpec((TB,), lambda i: (i,)),
            pl.BlockSpec((TB,), lambda i: (i,)),
        ],
        out_specs=pl.BlockSpec((GR, NV, TB), lambda i: (0, 0, i)),
        out_shape=jax.ShapeDtypeStruct((GR, NV, T), jnp.float32),
    )(idx0, idx1)


@functools.lru_cache(maxsize=None)
def _make_sc_gather(T):
    info = plsc.get_sparse_core_info()
    nw = info.num_cores * info.num_subcores
    t_per_w = T // nw
    mesh = plsc.VectorSubcoreMesh(core_axis_name="c", subcore_axis_name="s")

    @functools.partial(
        pl.kernel,
        mesh=mesh,
        out_type=jax.ShapeDtypeStruct((T, GR * DCODE), jnp.float32),
        scratch_types=[
            pltpu.VMEM((t_per_w,), jnp.int32),
            pltpu.VMEM((t_per_w,), jnp.int32),
            pltpu.VMEM((t_per_w, DCODE), jnp.float32),
            pltpu.VMEM((t_per_w, DCODE), jnp.float32),
            pltpu.SemaphoreType.DMA,
            pltpu.SemaphoreType.DMA,
        ],
    )
    def k(table_hbm, idx0_hbm, idx1_hbm, out_hbm, ia_v, ib_v, g0_v, g1_v,
          sem0, sem1):
        wid = lax.axis_index("s") * info.num_cores + lax.axis_index("c")
        base = wid * t_per_w
        pltpu.sync_copy(idx0_hbm.at[pl.ds(base, t_per_w)], ia_v)
        pltpu.sync_copy(idx1_hbm.at[pl.ds(base, t_per_w)], ib_v)
        c0 = pltpu.async_copy(table_hbm.at[ia_v], g0_v, sem0)
        c1 = pltpu.async_copy(table_hbm.at[ib_v], g1_v, sem1)
        c0.wait()
        pltpu.sync_copy(g0_v, out_hbm.at[pl.ds(base, t_per_w), pl.ds(0, DCODE)])
        c1.wait()
        pltpu.sync_copy(g1_v, out_hbm.at[pl.ds(base, t_per_w), pl.ds(DCODE, DCODE)])

    return k


def kernel(hidden_states, codevectors, w_proj, b_proj):
    B, S, H = hidden_states.shape
    T = B * S
    hs2d = hidden_states.reshape(T, H)
    w0 = w_proj[:, :NV]
    w1 = w_proj[:, NV:]
    b0 = b_proj[:NV].reshape(1, NV)
    b1 = b_proj[NV:].reshape(1, NV)
    idx0, idx1 = _argmax_call(hs2d, w0, w1, b0, b1)

    table = codevectors.reshape(GR * NV, DCODE)
    cv = _make_sc_gather(T)(table, idx0, idx1)
    cv = cv.reshape(B, S, GR * DCODE)
    dist_t = _dist_call(idx0, idx1)
    dist = jnp.transpose(dist_t, (2, 0, 1))
    return cv, dist


# speedup vs baseline: 1.1456x; 1.1456x over previous
import functools

import jax
import jax.numpy as jnp
from jax import lax
from jax.experimental import pallas as pl
from jax.experimental.pallas import tpu as pltpu
from jax.experimental.pallas import tpu_sc as plsc

DIM_H = 1024
NV = 320
GR = 2
DCODE = 128
TB = 512


def _argmax_body(hs_ref, w_ref, b_ref, idx0_ref, idx1_ref):
    hs = hs_ref[...]
    l = jnp.dot(hs, w_ref[...], preferred_element_type=jnp.float32)
    l = l + b_ref[...][None, :]
    lane = lax.broadcasted_iota(jnp.int32, (TB, 128), 1)
    big = jnp.int32(2 ** 30)
    ninf = jnp.float32(float("-inf"))
    c = [l[:, 128 * k:128 * (k + 1)] for k in range(5)]
    c2a = jnp.where(lane < 64, c[2], ninf)
    c2b = jnp.where(lane >= 64, c[2], ninf)

    m0 = jnp.max(jnp.maximum(jnp.maximum(c[0], c[1]), c2a), axis=1,
                 keepdims=True)
    m1 = jnp.max(jnp.maximum(jnp.maximum(c2b, c[3]), c[4]), axis=1,
                 keepdims=True)

    k0 = jnp.where(c[0] == m0, lane, big)
    k0 = jnp.minimum(k0, jnp.where(c[1] == m0, lane + 128, big))
    k0 = jnp.minimum(k0, jnp.where(c2a == m0, lane + 256, big))
    i0 = jnp.min(k0, axis=1, keepdims=True)

    k1 = jnp.where(c2b == m1, lane + 256, big)
    k1 = jnp.minimum(k1, jnp.where(c[3] == m1, lane + 384, big))
    k1 = jnp.minimum(k1, jnp.where(c[4] == m1, lane + 512, big))
    i1 = jnp.min(k1, axis=1, keepdims=True)

    idx0_ref[...] = i0
    idx1_ref[...] = i1


def _argmax_call(hs2d, w_proj, b_proj):
    T = hs2d.shape[0]
    return pl.pallas_call(
        _argmax_body,
        grid=(T // TB,),
        in_specs=[
            pl.BlockSpec((TB, DIM_H), lambda i: (i, 0)),
            pl.BlockSpec((DIM_H, GR * NV), lambda i: (0, 0)),
            pl.BlockSpec((GR * NV,), lambda i: (0,)),
        ],
        out_specs=[
            pl.BlockSpec((TB, 1), lambda i: (i, 0)),
            pl.BlockSpec((TB, 1), lambda i: (i, 0)),
        ],
        out_shape=[
            jax.ShapeDtypeStruct((T, 1), jnp.int32),
            jax.ShapeDtypeStruct((T, 1), jnp.int32),
        ],
    )(hs2d, w_proj, b_proj)


def _dist_body(idx0_ref, idx1_ref, dist_ref):
    iota = lax.broadcasted_iota(jnp.int32, (NV, TB), 0)
    i0 = idx0_ref[...]
    i1 = idx1_ref[...] - NV
    dist_ref[0] = (iota == i0[None, :]).astype(jnp.float32)
    dist_ref[1] = (iota == i1[None, :]).astype(jnp.float32)


def _dist_call(idx0, idx1):
    T = idx0.shape[0]
    return pl.pallas_call(
        _dist_body,
        grid=(T // TB,),
        in_specs=[
            pl.BlockSpec((TB,), lambda i: (i,)),
            pl.BlockSpec((TB,), lambda i: (i,)),
        ],
        out_specs=pl.BlockSpec((GR, NV, TB), lambda i: (0, 0, i)),
        out_shape=jax.ShapeDtypeStruct((GR, NV, T), jnp.float32),
    )(idx0, idx1)


@functools.lru_cache(maxsize=None)
def _make_sc_gather(T):
    info = plsc.get_sparse_core_info()
    nw = info.num_cores * info.num_subcores
    t_per_w = T // nw
    mesh = plsc.VectorSubcoreMesh(core_axis_name="c", subcore_axis_name="s")

    @functools.partial(
        pl.kernel,
        mesh=mesh,
        out_type=jax.ShapeDtypeStruct((T, GR * DCODE), jnp.float32),
        scratch_types=[
            pltpu.VMEM((t_per_w,), jnp.int32),
            pltpu.VMEM((t_per_w,), jnp.int32),
            pltpu.VMEM((t_per_w, DCODE), jnp.float32),
            pltpu.VMEM((t_per_w, DCODE), jnp.float32),
            pltpu.SemaphoreType.DMA,
            pltpu.SemaphoreType.DMA,
        ],
    )
    def k(table_hbm, idx0_hbm, idx1_hbm, out_hbm, ia_v, ib_v, g0_v, g1_v,
          sem0, sem1):
        wid = lax.axis_index("s") * info.num_cores + lax.axis_index("c")
        base = wid * t_per_w
        pltpu.sync_copy(idx0_hbm.at[pl.ds(base, t_per_w)], ia_v)
        pltpu.sync_copy(idx1_hbm.at[pl.ds(base, t_per_w)], ib_v)
        c0 = pltpu.async_copy(table_hbm.at[ia_v], g0_v, sem0)
        c1 = pltpu.async_copy(table_hbm.at[ib_v], g1_v, sem1)
        c0.wait()
        pltpu.sync_copy(g0_v, out_hbm.at[pl.ds(base, t_per_w), pl.ds(0, DCODE)])
        c1.wait()
        pltpu.sync_copy(g1_v, out_hbm.at[pl.ds(base, t_per_w), pl.ds(DCODE, DCODE)])

    return k


def kernel(hidden_states, codevectors, w_proj, b_proj):
    B, S, H = hidden_states.shape
    T = B * S
    hs2d = hidden_states.reshape(T, H)
    idx0, idx1 = _argmax_call(hs2d, w_proj, b_proj)
    idx0 = idx0.reshape(T)
    idx1 = idx1.reshape(T)

    table = codevectors.reshape(GR * NV, DCODE)
    cv = _make_sc_gather(T)(table, idx0, idx1)
    cv = cv.reshape(B, S, GR * DCODE)
    dist_t = _dist_call(idx0, idx1)
    dist = jnp.transpose(dist_t, (2, 0, 1))
    return cv, dist


# speedup vs baseline: 1.1823x; 1.0321x over previous
import functools

import jax
import jax.numpy as jnp
from jax import lax
from jax.experimental import pallas as pl
from jax.experimental.pallas import tpu as pltpu
from jax.experimental.pallas import tpu_sc as plsc

DIM_H = 1024
NV = 320
GR = 2
DCODE = 128
TB = 512


def _argmax_body(hs_ref, w_ref, b_ref, idx_ref):
    hs = hs_ref[...]
    l = jnp.dot(hs, w_ref[...], preferred_element_type=jnp.float32)
    l = l + b_ref[...][None, :]
    lane = lax.broadcasted_iota(jnp.int32, (TB, 128), 1)
    big = jnp.int32(2 ** 30)
    ninf = jnp.float32(float("-inf"))
    c = [l[:, 128 * k:128 * (k + 1)] for k in range(5)]
    c2a = jnp.where(lane < 64, c[2], ninf)
    c2b = jnp.where(lane >= 64, c[2], ninf)

    m0 = jnp.max(jnp.maximum(jnp.maximum(c[0], c[1]), c2a), axis=1,
                 keepdims=True)
    m1 = jnp.max(jnp.maximum(jnp.maximum(c2b, c[3]), c[4]), axis=1,
                 keepdims=True)

    k0 = jnp.where(c[0] == m0, lane, big)
    k0 = jnp.minimum(k0, jnp.where(c[1] == m0, lane + 128, big))
    k0 = jnp.minimum(k0, jnp.where(c2a == m0, lane + 256, big))
    i0 = jnp.min(k0, axis=1, keepdims=True)

    k1 = jnp.where(c2b == m1, lane + 256, big)
    k1 = jnp.minimum(k1, jnp.where(c[3] == m1, lane + 384, big))
    k1 = jnp.minimum(k1, jnp.where(c[4] == m1, lane + 512, big))
    i1 = jnp.min(k1, axis=1, keepdims=True)

    idx_ref[0] = i0
    idx_ref[1] = i1


def _argmax_call(hs2d, w_proj, b_proj):
    T = hs2d.shape[0]
    return pl.pallas_call(
        _argmax_body,
        grid=(T // TB,),
        in_specs=[
            pl.BlockSpec((TB, DIM_H), lambda i: (i, 0)),
            pl.BlockSpec((DIM_H, GR * NV), lambda i: (0, 0)),
            pl.BlockSpec((GR * NV,), lambda i: (0,)),
        ],
        out_specs=pl.BlockSpec((GR, TB, 1), lambda i: (0, i, 0)),
        out_shape=jax.ShapeDtypeStruct((GR, T, 1), jnp.int32),
    )(hs2d, w_proj, b_proj)


def _dist_body(idx_ref, dist_ref):
    iota = lax.broadcasted_iota(jnp.int32, (NV, TB), 0)
    i0 = idx_ref[0]
    i1 = idx_ref[1] - NV
    dist_ref[0] = (iota == i0[None, :]).astype(jnp.float32)
    dist_ref[1] = (iota == i1[None, :]).astype(jnp.float32)


def _dist_call(idx2):
    T = idx2.shape[1]
    return pl.pallas_call(
        _dist_body,
        grid=(T // TB,),
        in_specs=[pl.BlockSpec((GR, TB), lambda i: (0, i))],
        out_specs=pl.BlockSpec((GR, NV, TB), lambda i: (0, 0, i)),
        out_shape=jax.ShapeDtypeStruct((GR, NV, T), jnp.float32),
    )(idx2)


@functools.lru_cache(maxsize=None)
def _make_sc_gather(T):
    info = plsc.get_sparse_core_info()
    nw = info.num_cores * info.num_subcores
    t_per_w = T // nw
    mesh = plsc.VectorSubcoreMesh(core_axis_name="c", subcore_axis_name="s")

    @functools.partial(
        pl.kernel,
        mesh=mesh,
        out_type=jax.ShapeDtypeStruct((T, GR * DCODE), jnp.float32),
        scratch_types=[
            pltpu.VMEM((t_per_w,), jnp.int32),
            pltpu.VMEM((t_per_w,), jnp.int32),
            pltpu.VMEM((t_per_w, DCODE), jnp.float32),
            pltpu.VMEM((t_per_w, DCODE), jnp.float32),
            pltpu.SemaphoreType.DMA,
            pltpu.SemaphoreType.DMA,
            pltpu.SemaphoreType.DMA,
            pltpu.SemaphoreType.DMA,
        ],
    )
    def k(table_hbm, idx_hbm, out_hbm, ia_v, ib_v, g0_v, g1_v,
          sem0, sem1, sem2, sem3):
        wid = lax.axis_index("s") * info.num_cores + lax.axis_index("c")
        base = wid * t_per_w
        pltpu.sync_copy(idx_hbm.at[0, pl.ds(base, t_per_w)], ia_v)
        pltpu.sync_copy(idx_hbm.at[1, pl.ds(base, t_per_w)], ib_v)
        c0 = pltpu.async_copy(table_hbm.at[ia_v], g0_v, sem0)
        c1 = pltpu.async_copy(table_hbm.at[ib_v], g1_v, sem1)
        c0.wait()
        w0 = pltpu.async_copy(
            g0_v, out_hbm.at[pl.ds(base, t_per_w), pl.ds(0, DCODE)], sem2)
        c1.wait()
        w1 = pltpu.async_copy(
            g1_v, out_hbm.at[pl.ds(base, t_per_w), pl.ds(DCODE, DCODE)], sem3)
        w0.wait()
        w1.wait()

    return k


def kernel(hidden_states, codevectors, w_proj, b_proj):
    B, S, H = hidden_states.shape
    T = B * S
    hs2d = hidden_states.reshape(T, H)
    idx2 = _argmax_call(hs2d, w_proj, b_proj).reshape(GR, T)

    table = codevectors.reshape(GR * NV, DCODE)
    cv = _make_sc_gather(T)(table, idx2)
    cv = cv.reshape(B, S, GR * DCODE)
    dist_t = _dist_call(idx2)
    dist = jnp.transpose(dist_t, (2, 0, 1))
    return cv, dist


# speedup vs baseline: 1.2109x; 1.0241x over previous
import functools

import jax
import jax.numpy as jnp
from jax import lax
from jax.experimental import pallas as pl
from jax.experimental.pallas import tpu as pltpu
from jax.experimental.pallas import tpu_sc as plsc

DIM_H = 1024
NV = 320
GR = 2
DCODE = 128
TB = 1024


def _argmax_body(hs_ref, w_ref, b_ref, idx_ref):
    hs = hs_ref[...]
    l = jnp.dot(hs, w_ref[...], preferred_element_type=jnp.float32)
    l = l + b_ref[...][None, :]
    lane = lax.broadcasted_iota(jnp.int32, (TB, 128), 1)
    big = jnp.int32(2 ** 30)
    ninf = jnp.float32(float("-inf"))
    c = [l[:, 128 * k:128 * (k + 1)] for k in range(5)]
    c2a = jnp.where(lane < 64, c[2], ninf)
    c2b = jnp.where(lane >= 64, c[2], ninf)

    m0 = jnp.max(jnp.maximum(jnp.maximum(c[0], c[1]), c2a), axis=1,
                 keepdims=True)
    m1 = jnp.max(jnp.maximum(jnp.maximum(c2b, c[3]), c[4]), axis=1,
                 keepdims=True)

    k0 = jnp.where(c[0] == m0, lane, big)
    k0 = jnp.minimum(k0, jnp.where(c[1] == m0, lane + 128, big))
    k0 = jnp.minimum(k0, jnp.where(c2a == m0, lane + 256, big))
    i0 = jnp.min(k0, axis=1, keepdims=True)

    k1 = jnp.where(c2b == m1, lane + 256, big)
    k1 = jnp.minimum(k1, jnp.where(c[3] == m1, lane + 384, big))
    k1 = jnp.minimum(k1, jnp.where(c[4] == m1, lane + 512, big))
    i1 = jnp.min(k1, axis=1, keepdims=True)

    idx_ref[0] = i0
    idx_ref[1] = i1


def _argmax_call(hs2d, w_proj, b_proj):
    T = hs2d.shape[0]
    return pl.pallas_call(
        _argmax_body,
        grid=(T // TB,),
        in_specs=[
            pl.BlockSpec((TB, DIM_H), lambda i: (i, 0)),
            pl.BlockSpec((DIM_H, GR * NV), lambda i: (0, 0)),
            pl.BlockSpec((GR * NV,), lambda i: (0,)),
        ],
        out_specs=pl.BlockSpec((GR, TB, 1), lambda i: (0, i, 0)),
        out_shape=jax.ShapeDtypeStruct((GR, T, 1), jnp.int32),
    )(hs2d, w_proj, b_proj)


def _dist_body(idx_ref, dist_ref):
    iota = lax.broadcasted_iota(jnp.int32, (NV, TB), 0)
    i0 = idx_ref[0]
    i1 = idx_ref[1] - NV
    dist_ref[0] = (iota == i0[None, :]).astype(jnp.float32)
    dist_ref[1] = (iota == i1[None, :]).astype(jnp.float32)


def _dist_call(idx2):
    T = idx2.shape[1]
    return pl.pallas_call(
        _dist_body,
        grid=(T // TB,),
        in_specs=[pl.BlockSpec((GR, TB), lambda i: (0, i))],
        out_specs=pl.BlockSpec((GR, NV, TB), lambda i: (0, 0, i)),
        out_shape=jax.ShapeDtypeStruct((GR, NV, T), jnp.float32),
    )(idx2)


@functools.lru_cache(maxsize=None)
def _make_sc_gather(T):
    info = plsc.get_sparse_core_info()
    nw = info.num_cores * info.num_subcores
    t_per_w = T // nw
    mesh = plsc.VectorSubcoreMesh(core_axis_name="c", subcore_axis_name="s")

    @functools.partial(
        pl.kernel,
        mesh=mesh,
        out_type=jax.ShapeDtypeStruct((T, GR * DCODE), jnp.float32),
        scratch_types=[
            pltpu.VMEM((t_per_w,), jnp.int32),
            pltpu.VMEM((t_per_w,), jnp.int32),
            pltpu.VMEM((t_per_w, DCODE), jnp.float32),
            pltpu.VMEM((t_per_w, DCODE), jnp.float32),
            pltpu.SemaphoreType.DMA,
            pltpu.SemaphoreType.DMA,
            pltpu.SemaphoreType.DMA,
            pltpu.SemaphoreType.DMA,
        ],
    )
    def k(table_hbm, idx_hbm, out_hbm, ia_v, ib_v, g0_v, g1_v,
          sem0, sem1, sem2, sem3):
        wid = lax.axis_index("s") * info.num_cores + lax.axis_index("c")
        base = wid * t_per_w
        pltpu.sync_copy(idx_hbm.at[0, pl.ds(base, t_per_w)], ia_v)
        pltpu.sync_copy(idx_hbm.at[1, pl.ds(base, t_per_w)], ib_v)
        c0 = pltpu.async_copy(table_hbm.at[ia_v], g0_v, sem0)
        c1 = pltpu.async_copy(table_hbm.at[ib_v], g1_v, sem1)
        c0.wait()
        w0 = pltpu.async_copy(
            g0_v, out_hbm.at[pl.ds(base, t_per_w), pl.ds(0, DCODE)], sem2)
        c1.wait()
        w1 = pltpu.async_copy(
            g1_v, out_hbm.at[pl.ds(base, t_per_w), pl.ds(DCODE, DCODE)], sem3)
        w0.wait()
        w1.wait()

    return k


def kernel(hidden_states, codevectors, w_proj, b_proj):
    B, S, H = hidden_states.shape
    T = B * S
    hs2d = hidden_states.reshape(T, H)
    idx2 = _argmax_call(hs2d, w_proj, b_proj).reshape(GR, T)

    table = codevectors.reshape(GR * NV, DCODE)
    cv = _make_sc_gather(T)(table, idx2)
    cv = cv.reshape(B, S, GR * DCODE)
    dist_t = _dist_call(idx2)
    dist = jnp.transpose(dist_t, (2, 0, 1))
    return cv, dist
